# Initial kernel scaffold; baseline (speedup 1.0000x reference)
import jax
import jax.numpy as jnp
from jax.experimental import pallas as pl


def kernel(x, edge_index, batch, W0, b0, W1, b1, pool_w, Wd0, bd0, Wd1, bd1):
    raise NotImplementedError("write your pallas kernel here")


# speedup vs baseline: 24.3182x; 24.3182x over previous
import functools

import jax
import jax.numpy as jnp
from jax import lax
from jax.experimental import pallas as pl
from jax.experimental.pallas import tpu as pltpu
from jax.experimental.pallas import tpu_sc as plsc

N = 10000
NP = 10240
E = 320000
D = 128
K = 5000
NC = 2
NS = 16
NW = NC * NS
RPT = 80
PAD_PER_TILE = RPT * 128 - E // NW
RSUB = NP // NS

_f32 = jnp.float32


def _mesh():
    return plsc.VectorSubcoreMesh(
        core_axis_name="c", subcore_axis_name="s", num_cores=NC, num_subcores=NS
    )


def _segsum_body(d, table_hbm, src_hbm, dst_hbm, out_hbm, src_v, dst_v, rows,
                 zrows, gsem, acc):
    c = lax.axis_index("c")
    s = lax.axis_index("s")
    base = (c * NS + s) * RPT
    pltpu.sync_copy(src_hbm.at[pl.ds(base, RPT)], src_v)
    pltpu.sync_copy(dst_hbm.at[pl.ds(base, RPT)], dst_v)

    def zfill(i, _):
        if d == 1:
            zrows[pl.ds(i * 16, 16)] = jnp.zeros((16,), _f32)
        else:
            for j in range(d // 16):
                zrows[i, pl.ds(j * 16, 16)] = jnp.zeros((16,), _f32)
        return 0

    lax.fori_loop(0, 16, zfill, 0)

    def zacc(i, _):
        pltpu.sync_copy(zrows, acc.at[pl.ds(s * RSUB + i * 16, 16)])
        return 0

    lax.fori_loop(0, RSUB // 16, zacc, 0)
    plsc.subcore_barrier()

    def step(r, _):
        pltpu.async_copy(table_hbm.at[src_v.at[r]], rows, gsem).wait()
        pltpu.sync_copy(rows, acc.at[dst_v.at[r]], add=True)
        return 0

    lax.fori_loop(0, RPT, step, 0)
    plsc.subcore_barrier()
    pltpu.sync_copy(acc.at[pl.ds(s * RSUB, RSUB)],
                    out_hbm.at[c, pl.ds(s * RSUB, RSUB)])


def _make_segsum(d):
    if d == 1:
        tshape, rshape, zshape, ashape, oshape = (
            (NP,), (128,), (16,), (NP,), (NC, NP))
    else:
        tshape, rshape, zshape, ashape, oshape = (
            (NP, d), (128, d), (16, d), (NP, d), (NC, NP, d))
    return pl.kernel(
        functools.partial(_segsum_body, d),
        out_type=jax.ShapeDtypeStruct(oshape, _f32),
        mesh=_mesh(),
        scratch_types=[
            pltpu.VMEM((RPT, 128), jnp.int32),
            pltpu.VMEM((RPT, 128), jnp.int32),
            pltpu.VMEM(rshape, _f32),
            pltpu.VMEM(zshape, _f32),
            pltpu.SemaphoreType.DMA,
            pltpu.VMEM_SHARED(ashape, _f32),
        ],
    )


def _tc1(cnt_ref, x_ref, w0_ref, t0_ref, xw0_ref, dinv_ref):
    cnt = cnt_ref[0] + cnt_ref[1]
    dinv = lax.rsqrt(cnt + 1.0)
    xw0 = jnp.dot(x_ref[...], w0_ref[...], preferred_element_type=_f32)
    t0_ref[...] = xw0 * dinv
    xw0_ref[...] = xw0
    dinv_ref[...] = dinv


def _tc2(s_ref, xw0_ref, dinv_ref, b0_ref, pw_ref, w1_ref,
         h0_ref, sel_ref, xw1_ref):
    S = s_ref[0] + s_ref[1]
    dinv = dinv_ref[...]
    validN = lax.broadcasted_iota(jnp.int32, (NP, 1), 0) < N
    h0 = jnp.maximum(S * dinv + xw0_ref[...] * (dinv * dinv) + b0_ref[...],
                     0.0)
    h0 = jnp.where(validN, h0, 0.0)
    h0_ref[...] = h0

    pw = pw_ref[...]
    pwn = pw * lax.rsqrt(jnp.sum(pw * pw))
    h3 = h0.reshape(NP // 128, 128, D)
    z = jnp.sum(h3 * pwn.reshape(1, 1, D), axis=2)
    score = jnp.tanh(z)

    u = lax.bitcast_convert_type(score, jnp.int32)
    key = u ^ (lax.shift_right_arithmetic(u, 31) & jnp.int32(0x7FFFFFFF))
    fi = (lax.broadcasted_iota(jnp.int32, (NP // 128, 128), 0) * 128
          + lax.broadcasted_iota(jnp.int32, (NP // 128, 128), 1))
    valid = fi < N
    key = jnp.where(valid, key, jnp.int32(-(2 ** 31)))

    def bis(_, lohi):
        lo, hi = lohi
        mid = (lax.shift_right_arithmetic(lo, 1)
               + lax.shift_right_arithmetic(hi, 1) + (lo & hi & 1))
        cnt = jnp.sum((key >= mid).astype(jnp.int32))
        ge = cnt >= K
        return jnp.where(ge, mid, lo), jnp.where(ge, hi, mid)

    lo, _hi = lax.fori_loop(
        0, 32, bis, (jnp.int32(-(2 ** 31)), jnp.int32(2 ** 31 - 1)))
    t = lo
    c_gt = jnp.sum((key > t).astype(jnp.int32))
    need = (K - c_gt).astype(_f32)

    tie = ((key == t) & valid).astype(_f32)
    iu = lax.broadcasted_iota(jnp.int32, (128, 128), 0)
    ju = lax.broadcasted_iota(jnp.int32, (128, 128), 1)
    su = (iu < ju).astype(_f32)
    rank_in = jnp.dot(tie, su, preferred_element_type=_f32)
    rowsum = jnp.sum(tie, axis=1, keepdims=True)
    ir = lax.broadcasted_iota(jnp.int32, (NP // 128, NP // 128), 0)
    jr = lax.broadcasted_iota(jnp.int32, (NP // 128, NP // 128), 1)
    sl = (jr < ir).astype(_f32)
    rowoff = jnp.dot(sl, rowsum, preferred_element_type=_f32)
    rank = rank_in + rowoff

    selb = (key > t) | ((key == t) & valid & (rank < need))
    self32 = selb.astype(_f32)
    sel_ref[...] = self32
    g = score * self32
    xp = (h3 * g[:, :, None]).reshape(NP, D)
    xw1_ref[...] = jnp.dot(xp, w1_ref[...], preferred_element_type=_f32)


def _tc3(pcnt_ref, sel_ref, xw1_ref, t1_ref, pdinv_ref):
    pc = (pcnt_ref[0] + pcnt_ref[1]) * sel_ref[...]
    pdinv = lax.rsqrt(pc + 1.0)
    pdinv_ref[...] = pdinv
    t1_ref[...] = xw1_ref[...] * pdinv


def _tc4(s_ref, xw1_ref, pdinv_ref, sel_ref, b1_ref, wd0_ref,
         t2_ref, xwd0_ref):
    S = s_ref[0] + s_ref[1]
    pdinv = pdinv_ref[...]
    sel = sel_ref[...]
    h1 = jnp.maximum(
        S * pdinv * sel + xw1_ref[...] * (pdinv * pdinv) + b1_ref[...],
        0.0) * sel
    wd0c = wd0_ref[0] + wd0_ref[1]
    xwd0 = jnp.dot(h1, wd0c, preferred_element_type=_f32)
    xwd0_ref[...] = xwd0
    t2_ref[...] = xwd0 * pdinv


def _tc5(s_ref, xwd0_ref, pdinv_ref, sel_ref, bd0_ref, wd1_ref, h0_ref,
         dinv_ref, t3_ref, xwd1_ref):
    S = s_ref[0] + s_ref[1]
    pdinv = pdinv_ref[...]
    sel = sel_ref[...]
    d0 = jnp.maximum(
        S * pdinv * sel + xwd0_ref[...] * (pdinv * pdinv) + bd0_ref[...],
        0.0) * sel
    xwd1 = (jnp.dot(d0, wd1_ref[0], preferred_element_type=_f32)
            + jnp.dot(h0_ref[...], wd1_ref[1], preferred_element_type=_f32))
    xwd1_ref[...] = xwd1
    t3_ref[...] = xwd1 * dinv_ref[...]


def _tc6(s_ref, xwd1_ref, dinv_ref, bd1_ref, out_ref):
    S = s_ref[0] + s_ref[1]
    dinv = dinv_ref[...]
    out_ref[...] = S * dinv + xwd1_ref[...] * (dinv * dinv) + bd1_ref[...]


def _tc(body, out_shapes, *args):
    return pl.pallas_call(
        body,
        out_shape=[jax.ShapeDtypeStruct(s, _f32) for s in out_shapes],
        compiler_params=pltpu.CompilerParams(vmem_limit_bytes=100 * 2**20),
    )(*args)


def kernel(x, edge_index, batch, W0, b0, W1, b1, pool_w, Wd0, bd0, Wd1, bd1):
    src = edge_index[0].astype(jnp.int32)
    dst = edge_index[1].astype(jnp.int32)
    padi = jnp.arange(NW * PAD_PER_TILE, dtype=jnp.int32)
    padr = N + (padi % (NP - N))
    srcE = jnp.concatenate(
        [src.reshape(NW, E // NW), padr.reshape(NW, PAD_PER_TILE)],
        axis=1).reshape(NW * RPT, 128)
    dstE = jnp.concatenate(
        [dst.reshape(NW, E // NW), padr.reshape(NW, PAD_PER_TILE)],
        axis=1).reshape(NW * RPT, 128)

    xP = jnp.zeros((NP, D), _f32).at[:N].set(x.astype(_f32))
    ones_sel = jnp.zeros((NP,), _f32).at[:N].set(1.0)

    count = _make_segsum(1)
    segsum = _make_segsum(D)

    cnt = count(ones_sel, srcE, dstE).reshape(NC, NP, 1)
    t0, xw0, dinv = _tc(_tc1, [(NP, D), (NP, D), (NP, 1)], cnt, xP, W0)
    S0 = segsum(t0, srcE, dstE)
    h0, sel80, xw1 = _tc(
        _tc2, [(NP, D), (NP // 128, 128), (NP, D)],
        S0, xw0, dinv, b0.reshape(1, D), pool_w.reshape(1, D), W1)
    selN = sel80.reshape(NP)
    pcnt = count(selN, srcE, dstE).reshape(NC, NP, 1)
    t1, pdinv = _tc(_tc3, [(NP, D), (NP, 1)], pcnt, selN.reshape(NP, 1), xw1)
    S1 = segsum(t1, srcE, dstE)
    t2, xwd0 = _tc(
        _tc4, [(NP, D), (NP, D)],
        S1, xw1, pdinv, selN.reshape(NP, 1), b1.reshape(1, D),
        Wd0.reshape(2, D, D))
    S2 = segsum(t2, srcE, dstE)
    t3, xwd1 = _tc(
        _tc5, [(NP, D), (NP, D)],
        S2, xwd0, pdinv, selN.reshape(NP, 1), bd0.reshape(1, D),
        Wd1.reshape(2, D, D), h0, dinv)
    S3 = segsum(t3, srcE, dstE)
    (out,) = _tc(_tc6, [(NP, D)], S3, xwd1, dinv, bd1.reshape(1, D))
    return out[:N]


# speedup vs baseline: 31.5562x; 1.2976x over previous
import functools

import jax
import jax.numpy as jnp
from jax import lax
from jax.experimental import pallas as pl
from jax.experimental.pallas import tpu as pltpu
from jax.experimental.pallas import tpu_sc as plsc

N = 10000
NP = 10240
E = 320000
D = 128
K = 5000
NC = 2
NS = 16
NW = NC * NS
RPT = 80
PAD_PER_TILE = RPT * 128 - E // NW
RSUB = NP // NS

_f32 = jnp.float32


def _mesh():
    return plsc.VectorSubcoreMesh(
        core_axis_name="c", subcore_axis_name="s", num_cores=NC, num_subcores=NS
    )


def _decode(packed_v, row, half, dec):
    for j in range(4):
        v = packed_v[row, pl.ds(half * 64 + j * 16, 16)]
        dec[pl.ds(j * 32, 16)] = v & jnp.int32(0xFFFF)
        dec[pl.ds(j * 32 + 16, 16)] = lax.shift_right_logical(v, 16)


def _segsum_body(d, table_hbm, src_hbm, dst_hbm, out_hbm, srcp, dstp, rows0,
                 rows1, zrows, sdec0, sdec1, ddec, sem0, sem1, acc):
    c = lax.axis_index("c")
    s = lax.axis_index("s")
    base = (c * NS + s) * (RPT // 2)
    pltpu.sync_copy(src_hbm.at[pl.ds(base, RPT // 2)], srcp)
    pltpu.sync_copy(dst_hbm.at[pl.ds(base, RPT // 2)], dstp)

    _decode(srcp, 0, 0, sdec0)
    pltpu.async_copy(table_hbm.at[sdec0], rows0, sem0)
    _decode(srcp, 0, 1, sdec1)
    pltpu.async_copy(table_hbm.at[sdec1], rows1, sem1)

    def zfill(i, _):
        if d == 1:
            zrows[pl.ds(i * 16, 16)] = jnp.zeros((16,), _f32)
        else:
            for j in range(d // 16):
                zrows[i, pl.ds(j * 16, 16)] = jnp.zeros((16,), _f32)
        return 0

    lax.fori_loop(0, 16, zfill, 0)

    def zacc(i, _):
        pltpu.sync_copy(zrows, acc.at[pl.ds(s * RSUB + i * 16, 16)])
        return 0

    lax.fori_loop(0, RSUB // 16, zacc, 0)
    plsc.subcore_barrier()

    def step(g, _):
        pltpu.make_async_copy(table_hbm.at[sdec0], rows0, sem0).wait()
        _decode(dstp, g, 0, ddec)
        pltpu.sync_copy(rows0, acc.at[ddec], add=True)

        @pl.when(g + 1 < RPT // 2)
        def _():
            _decode(srcp, g + 1, 0, sdec0)
            pltpu.async_copy(table_hbm.at[sdec0], rows0, sem0)

        pltpu.make_async_copy(table_hbm.at[sdec1], rows1, sem1).wait()
        _decode(dstp, g, 1, ddec)
        pltpu.sync_copy(rows1, acc.at[ddec], add=True)

        @pl.when(g + 1 < RPT // 2)
        def _():
            _decode(srcp, g + 1, 1, sdec1)
            pltpu.async_copy(table_hbm.at[sdec1], rows1, sem1)

        return 0

    lax.fori_loop(0, RPT // 2, step, 0)
    plsc.subcore_barrier()
    pltpu.sync_copy(acc.at[pl.ds(s * RSUB, RSUB)],
                    out_hbm.at[c, pl.ds(s * RSUB, RSUB)])


def _make_segsum(d):
    if d == 1:
        tshape, rshape, zshape, ashape, oshape = (
            (NP,), (128,), (16,), (NP,), (NC, NP))
    else:
        tshape, rshape, zshape, ashape, oshape = (
            (NP, d), (128, d), (16, d), (NP, d), (NC, NP, d))
    return pl.kernel(
        functools.partial(_segsum_body, d),
        out_type=jax.ShapeDtypeStruct(oshape, _f32),
        mesh=_mesh(),
        scratch_types=[
            pltpu.VMEM((RPT // 2, 128), jnp.int32),
            pltpu.VMEM((RPT // 2, 128), jnp.int32),
            pltpu.VMEM(rshape, _f32),
            pltpu.VMEM(rshape, _f32),
            pltpu.VMEM(zshape, _f32),
            pltpu.VMEM((128,), jnp.int32),
            pltpu.VMEM((128,), jnp.int32),
            pltpu.VMEM((128,), jnp.int32),
            pltpu.SemaphoreType.DMA,
            pltpu.SemaphoreType.DMA,
            pltpu.VMEM_SHARED(ashape, _f32),
        ],
    )


def _tc1(cnt_ref, x_ref, w0_ref, t0_ref, xw0_ref, dinv_ref):
    cnt = cnt_ref[0] + cnt_ref[1]
    dinv = lax.rsqrt(cnt + 1.0)
    xw0 = jnp.dot(x_ref[...], w0_ref[...], preferred_element_type=_f32)
    t0_ref[...] = xw0 * dinv
    xw0_ref[...] = xw0
    dinv_ref[...] = dinv


def _tc2(s_ref, xw0_ref, dinv_ref, b0_ref, pw_ref, w1_ref,
         h0_ref, sel_ref, xw1_ref):
    S = s_ref[0] + s_ref[1]
    dinv = dinv_ref[...]
    validN = lax.broadcasted_iota(jnp.int32, (NP, 1), 0) < N
    h0 = jnp.maximum(S * dinv + xw0_ref[...] * (dinv * dinv) + b0_ref[...],
                     0.0)
    h0 = jnp.where(validN, h0, 0.0)
    h0_ref[...] = h0

    pw = pw_ref[...]
    pwn = pw * lax.rsqrt(jnp.sum(pw * pw))
    h3 = h0.reshape(NP // 128, 128, D)
    z = jnp.sum(h3 * pwn.reshape(1, 1, D), axis=2)
    score = jnp.tanh(z)

    u = lax.bitcast_convert_type(score, jnp.int32)
    key = u ^ (lax.shift_right_arithmetic(u, 31) & jnp.int32(0x7FFFFFFF))
    fi = (lax.broadcasted_iota(jnp.int32, (NP // 128, 128), 0) * 128
          + lax.broadcasted_iota(jnp.int32, (NP // 128, 128), 1))
    valid = fi < N
    key = jnp.where(valid, key, jnp.int32(-(2 ** 31)))

    def bis(_, lohi):
        lo, hi = lohi
        mid = (lax.shift_right_arithmetic(lo, 1)
               + lax.shift_right_arithmetic(hi, 1) + (lo & hi & 1))
        cnt = jnp.sum((key >= mid).astype(jnp.int32))
        ge = cnt >= K
        return jnp.where(ge, mid, lo), jnp.where(ge, hi, mid)

    lo, _hi = lax.fori_loop(
        0, 32, bis, (jnp.int32(-(2 ** 31)), jnp.int32(2 ** 31 - 1)))
    t = lo
    c_gt = jnp.sum((key > t).astype(jnp.int32))
    need = (K - c_gt).astype(_f32)

    tie = ((key == t) & valid).astype(_f32)
    iu = lax.broadcasted_iota(jnp.int32, (128, 128), 0)
    ju = lax.broadcasted_iota(jnp.int32, (128, 128), 1)
    su = (iu < ju).astype(_f32)
    rank_in = jnp.dot(tie, su, preferred_element_type=_f32)
    rowsum = jnp.sum(tie, axis=1, keepdims=True)
    ir = lax.broadcasted_iota(jnp.int32, (NP // 128, NP // 128), 0)
    jr = lax.broadcasted_iota(jnp.int32, (NP // 128, NP // 128), 1)
    sl = (jr < ir).astype(_f32)
    rowoff = jnp.dot(sl, rowsum, preferred_element_type=_f32)
    rank = rank_in + rowoff

    selb = (key > t) | ((key == t) & valid & (rank < need))
    self32 = selb.astype(_f32)
    sel_ref[...] = self32
    g = score * self32
    xp = (h3 * g[:, :, None]).reshape(NP, D)
    xw1_ref[...] = jnp.dot(xp, w1_ref[...], preferred_element_type=_f32)


def _tc3(pcnt_ref, sel_ref, xw1_ref, t1_ref, pdinv_ref):
    pc = (pcnt_ref[0] + pcnt_ref[1]) * sel_ref[...]
    pdinv = lax.rsqrt(pc + 1.0)
    pdinv_ref[...] = pdinv
    t1_ref[...] = xw1_ref[...] * pdinv


def _tc4(s_ref, xw1_ref, pdinv_ref, sel_ref, b1_ref, wd0_ref,
         t2_ref, xwd0_ref):
    S = s_ref[0] + s_ref[1]
    pdinv = pdinv_ref[...]
    sel = sel_ref[...]
    h1 = jnp.maximum(
        S * pdinv * sel + xw1_ref[...] * (pdinv * pdinv) + b1_ref[...],
        0.0) * sel
    wd0c = wd0_ref[0] + wd0_ref[1]
    xwd0 = jnp.dot(h1, wd0c, preferred_element_type=_f32)
    xwd0_ref[...] = xwd0
    t2_ref[...] = xwd0 * pdinv


def _tc5(s_ref, xwd0_ref, pdinv_ref, sel_ref, bd0_ref, wd1_ref, h0_ref,
         dinv_ref, t3_ref, xwd1_ref):
    S = s_ref[0] + s_ref[1]
    pdinv = pdinv_ref[...]
    sel = sel_ref[...]
    d0 = jnp.maximum(
        S * pdinv * sel + xwd0_ref[...] * (pdinv * pdinv) + bd0_ref[...],
        0.0) * sel
    xwd1 = (jnp.dot(d0, wd1_ref[0], preferred_element_type=_f32)
            + jnp.dot(h0_ref[...], wd1_ref[1], preferred_element_type=_f32))
    xwd1_ref[...] = xwd1
    t3_ref[...] = xwd1 * dinv_ref[...]


def _tc6(s_ref, xwd1_ref, dinv_ref, bd1_ref, out_ref):
    S = s_ref[0] + s_ref[1]
    dinv = dinv_ref[...]
    out_ref[...] = S * dinv + xwd1_ref[...] * (dinv * dinv) + bd1_ref[...]


def _tc(body, out_shapes, *args):
    return pl.pallas_call(
        body,
        out_shape=[jax.ShapeDtypeStruct(s, _f32) for s in out_shapes],
        compiler_params=pltpu.CompilerParams(vmem_limit_bytes=100 * 2**20),
    )(*args)


def kernel(x, edge_index, batch, W0, b0, W1, b1, pool_w, Wd0, bd0, Wd1, bd1):
    src = edge_index[0].astype(jnp.int32)
    dst = edge_index[1].astype(jnp.int32)
    padi = jnp.arange(NW * PAD_PER_TILE, dtype=jnp.int32)
    padr = N + (padi % (NP - N))

    def pack_idx(a):
        a = jnp.concatenate(
            [a.reshape(NW, E // NW), padr.reshape(NW, PAD_PER_TILE)],
            axis=1)
        p = a[:, 0::2] | (a[:, 1::2] << 16)
        return p.reshape(NW * RPT // 2, 128)

    srcE = pack_idx(src)
    dstE = pack_idx(dst)

    xP = jnp.zeros((NP, D), _f32).at[:N].set(x.astype(_f32))
    ones_sel = jnp.zeros((NP,), _f32).at[:N].set(1.0)

    count = _make_segsum(1)
    segsum = _make_segsum(D)

    cnt = count(ones_sel, srcE, dstE).reshape(NC, NP, 1)
    t0, xw0, dinv = _tc(_tc1, [(NP, D), (NP, D), (NP, 1)], cnt, xP, W0)
    S0 = segsum(t0, srcE, dstE)
    h0, sel80, xw1 = _tc(
        _tc2, [(NP, D), (NP // 128, 128), (NP, D)],
        S0, xw0, dinv, b0.reshape(1, D), pool_w.reshape(1, D), W1)
    selN = sel80.reshape(NP)
    pcnt = count(selN, srcE, dstE).reshape(NC, NP, 1)
    t1, pdinv = _tc(_tc3, [(NP, D), (NP, 1)], pcnt, selN.reshape(NP, 1), xw1)
    S1 = segsum(t1, srcE, dstE)
    t2, xwd0 = _tc(
        _tc4, [(NP, D), (NP, D)],
        S1, xw1, pdinv, selN.reshape(NP, 1), b1.reshape(1, D),
        Wd0.reshape(2, D, D))
    S2 = segsum(t2, srcE, dstE)
    t3, xwd1 = _tc(
        _tc5, [(NP, D), (NP, D)],
        S2, xwd0, pdinv, selN.reshape(NP, 1), bd0.reshape(1, D),
        Wd1.reshape(2, D, D), h0, dinv)
    S3 = segsum(t3, srcE, dstE)
    (out,) = _tc(_tc6, [(NP, D)], S3, xwd1, dinv, bd1.reshape(1, D))
    return out[:N]


# speedup vs baseline: 34.9366x; 1.1071x over previous
import functools

import jax
import jax.numpy as jnp
from jax import lax
from jax.experimental import pallas as pl
from jax.experimental.pallas import tpu as pltpu
from jax.experimental.pallas import tpu_sc as plsc

N = 10000
NP = 10240
E = 320000
D = 128
K = 5000
NC = 2
NS = 16
NW = NC * NS
RPT = 80
PAD_PER_TILE = RPT * 128 - E // NW
RSUB = NP // NS

_f32 = jnp.float32


def _mesh():
    return plsc.VectorSubcoreMesh(
        core_axis_name="c", subcore_axis_name="s", num_cores=NC, num_subcores=NS
    )


def _decode(packed_v, row, half, dec):
    for j in range(4):
        v = packed_v[row, pl.ds(half * 64 + j * 16, 16)]
        dec[pl.ds(j * 32, 16)] = v & jnp.int32(0xFFFF)
        dec[pl.ds(j * 32 + 16, 16)] = lax.shift_right_logical(v, 16)


def _segsum_body(d, table_hbm, src_hbm, dst_hbm, out_hbm, srcp, dstp, rows0,
                 rows1, zrows, sdec0, sdec1, ddec, sem0, sem1, acc):
    c = lax.axis_index("c")
    s = lax.axis_index("s")
    base = (c * NS + s) * (RPT // 2)
    pltpu.sync_copy(src_hbm.at[pl.ds(base, RPT // 2)], srcp)
    pltpu.sync_copy(dst_hbm.at[pl.ds(base, RPT // 2)], dstp)

    _decode(srcp, 0, 0, sdec0)
    pltpu.async_copy(table_hbm.at[sdec0], rows0, sem0)
    _decode(srcp, 0, 1, sdec1)
    pltpu.async_copy(table_hbm.at[sdec1], rows1, sem1)

    def zfill(i, _):
        if d == 1:
            zrows[pl.ds(i * 16, 16)] = jnp.zeros((16,), _f32)
        else:
            for j in range(d // 16):
                zrows[i, pl.ds(j * 16, 16)] = jnp.zeros((16,), _f32)
        return 0

    lax.fori_loop(0, 16, zfill, 0)

    def zacc(i, _):
        pltpu.sync_copy(zrows, acc.at[pl.ds(s * RSUB + i * 16, 16)])
        return 0

    lax.fori_loop(0, RSUB // 16, zacc, 0)
    plsc.subcore_barrier()

    def step(g, _):
        pltpu.make_async_copy(table_hbm.at[sdec0], rows0, sem0).wait()
        _decode(dstp, g, 0, ddec)
        pltpu.sync_copy(rows0, acc.at[ddec], add=True)

        @pl.when(g + 1 < RPT // 2)
        def _():
            _decode(srcp, g + 1, 0, sdec0)
            pltpu.async_copy(table_hbm.at[sdec0], rows0, sem0)

        pltpu.make_async_copy(table_hbm.at[sdec1], rows1, sem1).wait()
        _decode(dstp, g, 1, ddec)
        pltpu.sync_copy(rows1, acc.at[ddec], add=True)

        @pl.when(g + 1 < RPT // 2)
        def _():
            _decode(srcp, g + 1, 1, sdec1)
            pltpu.async_copy(table_hbm.at[sdec1], rows1, sem1)

        return 0

    lax.fori_loop(0, RPT // 2, step, 0)
    plsc.subcore_barrier()
    pltpu.sync_copy(acc.at[pl.ds(s * RSUB, RSUB)],
                    out_hbm.at[c, pl.ds(s * RSUB, RSUB)])


def _ones_count_body(dst_hbm, out_hbm, dstp, ones_v, zv, ddec, acc):
    c = lax.axis_index("c")
    s = lax.axis_index("s")
    base = (c * NS + s) * (RPT // 2)
    pltpu.sync_copy(dst_hbm.at[pl.ds(base, RPT // 2)], dstp)

    for j in range(8):
        ones_v[pl.ds(j * 16, 16)] = jnp.ones((16,), _f32)
        zv[pl.ds(j * 16, 16)] = jnp.zeros((16,), _f32)

    def zfill(i, _):
        zv[pl.ds(i * 16, 16)] = jnp.zeros((16,), _f32)
        return 0

    lax.fori_loop(8, RSUB // 16, zfill, 0)
    pltpu.sync_copy(zv, acc.at[pl.ds(s * RSUB, RSUB)])
    plsc.subcore_barrier()

    def step(g, _):
        _decode(dstp, g, 0, ddec)
        pltpu.sync_copy(ones_v, acc.at[ddec], add=True)
        _decode(dstp, g, 1, ddec)
        pltpu.sync_copy(ones_v, acc.at[ddec], add=True)
        return 0

    lax.fori_loop(0, RPT // 2, step, 0)
    plsc.subcore_barrier()
    pltpu.sync_copy(acc.at[pl.ds(s * RSUB, RSUB)],
                    out_hbm.at[c, pl.ds(s * RSUB, RSUB)])


def _make_ones_count():
    return pl.kernel(
        _ones_count_body,
        out_type=jax.ShapeDtypeStruct((NC, NP), _f32),
        mesh=_mesh(),
        scratch_types=[
            pltpu.VMEM((RPT // 2, 128), jnp.int32),
            pltpu.VMEM((128,), _f32),
            pltpu.VMEM((RSUB,), _f32),
            pltpu.VMEM((128,), jnp.int32),
            pltpu.VMEM_SHARED((NP,), _f32),
        ],
    )


def _make_segsum(d):
    if d == 1:
        rshape, zshape, ashape, oshape = ((128,), (16,), (NP,), (NC, NP))
    else:
        rshape, zshape, ashape, oshape = (
            (128, d), (16, d), (NP, d), (NC, NP, d))
    return pl.kernel(
        functools.partial(_segsum_body, d),
        out_type=jax.ShapeDtypeStruct(oshape, _f32),
        mesh=_mesh(),
        scratch_types=[
            pltpu.VMEM((RPT // 2, 128), jnp.int32),
            pltpu.VMEM((RPT // 2, 128), jnp.int32),
            pltpu.VMEM(rshape, _f32),
            pltpu.VMEM(rshape, _f32),
            pltpu.VMEM(zshape, _f32),
            pltpu.VMEM((128,), jnp.int32),
            pltpu.VMEM((128,), jnp.int32),
            pltpu.VMEM((128,), jnp.int32),
            pltpu.SemaphoreType.DMA,
            pltpu.SemaphoreType.DMA,
            pltpu.VMEM_SHARED(ashape, _f32),
        ],
    )


def _tc1(cnt_ref, x_ref, w0_ref, t0_ref, dinv_ref):
    cnt = cnt_ref[0] + cnt_ref[1]
    dinv = lax.rsqrt(cnt + 1.0)
    xw0 = jnp.dot(x_ref[...], w0_ref[...], preferred_element_type=_f32)
    t0_ref[...] = xw0 * dinv
    dinv_ref[...] = dinv


def _tc2(s_ref, t0_ref, dinv_ref, b0_ref, pw_ref, w1_ref,
         h0_ref, sel_ref, xw1_ref):
    S = s_ref[0] + s_ref[1]
    dinv = dinv_ref[...]
    validN = lax.broadcasted_iota(jnp.int32, (NP, 1), 0) < N
    h0 = jnp.maximum((S + t0_ref[...]) * dinv + b0_ref[...], 0.0)
    h0 = jnp.where(validN, h0, 0.0)
    h0_ref[...] = h0

    pw = pw_ref[...]
    pwn = pw * lax.rsqrt(jnp.sum(pw * pw))
    h3 = h0.reshape(NP // 128, 128, D)
    z = jnp.sum(h3 * pwn.reshape(1, 1, D), axis=2)
    score = jnp.tanh(z)

    u = lax.bitcast_convert_type(score, jnp.int32)
    key = u ^ (lax.shift_right_arithmetic(u, 31) & jnp.int32(0x7FFFFFFF))
    fi = (lax.broadcasted_iota(jnp.int32, (NP // 128, 128), 0) * 128
          + lax.broadcasted_iota(jnp.int32, (NP // 128, 128), 1))
    valid = fi < N
    key = jnp.where(valid, key, jnp.int32(-(2 ** 31)))

    def bis(_, lohi):
        lo, hi = lohi
        mid = (lax.shift_right_arithmetic(lo, 1)
               + lax.shift_right_arithmetic(hi, 1) + (lo & hi & 1))
        cnt = jnp.sum((key >= mid).astype(jnp.int32))
        ge = cnt >= K
        return jnp.where(ge, mid, lo), jnp.where(ge, hi, mid)

    lo, _hi = lax.fori_loop(
        0, 32, bis, (jnp.int32(-(2 ** 31)), jnp.int32(2 ** 31 - 1)))
    t = lo
    c_gt = jnp.sum((key > t).astype(jnp.int32))
    need = (K - c_gt).astype(_f32)

    tie = ((key == t) & valid).astype(_f32)
    iu = lax.broadcasted_iota(jnp.int32, (128, 128), 0)
    ju = lax.broadcasted_iota(jnp.int32, (128, 128), 1)
    su = (iu < ju).astype(_f32)
    rank_in = jnp.dot(tie, su, preferred_element_type=_f32)
    rowsum = jnp.sum(tie, axis=1, keepdims=True)
    ir = lax.broadcasted_iota(jnp.int32, (NP // 128, NP // 128), 0)
    jr = lax.broadcasted_iota(jnp.int32, (NP // 128, NP // 128), 1)
    sl = (jr < ir).astype(_f32)
    rowoff = jnp.dot(sl, rowsum, preferred_element_type=_f32)
    rank = rank_in + rowoff

    selb = (key > t) | ((key == t) & valid & (rank < need))
    self32 = selb.astype(_f32)
    sel_ref[...] = self32
    g = score * self32
    xp = (h3 * g[:, :, None]).reshape(NP, D)
    xw1_ref[...] = jnp.dot(xp, w1_ref[...], preferred_element_type=_f32)


def _tc3(pcnt_ref, sel_ref, xw1_ref, t1_ref, pdinv_ref):
    pc = (pcnt_ref[0] + pcnt_ref[1]) * sel_ref[...]
    pdinv = lax.rsqrt(pc + 1.0)
    pdinv_ref[...] = pdinv
    t1_ref[...] = xw1_ref[...] * pdinv


def _tc4(s_ref, t1_ref, pdinv_ref, sel_ref, b1_ref, wd0_ref, t2_ref):
    S = s_ref[0] + s_ref[1]
    pdinv = pdinv_ref[...]
    sel = sel_ref[...]
    h1 = jnp.maximum(
        (S * sel + t1_ref[...]) * pdinv + b1_ref[...], 0.0) * sel
    wd0c = wd0_ref[0] + wd0_ref[1]
    xwd0 = jnp.dot(h1, wd0c, preferred_element_type=_f32)
    t2_ref[...] = xwd0 * pdinv


def _tc5(s_ref, t2_ref, pdinv_ref, sel_ref, bd0_ref, wd1_ref, h0_ref,
         dinv_ref, t3_ref):
    S = s_ref[0] + s_ref[1]
    pdinv = pdinv_ref[...]
    sel = sel_ref[...]
    d0 = jnp.maximum(
        (S * sel + t2_ref[...]) * pdinv + bd0_ref[...], 0.0) * sel
    xwd1 = (jnp.dot(d0, wd1_ref[0], preferred_element_type=_f32)
            + jnp.dot(h0_ref[...], wd1_ref[1], preferred_element_type=_f32))
    t3_ref[...] = xwd1 * dinv_ref[...]


def _tc6(s_ref, t3_ref, dinv_ref, bd1_ref, out_ref):
    S = s_ref[0] + s_ref[1]
    dinv = dinv_ref[...]
    out_ref[...] = (S + t3_ref[...]) * dinv + bd1_ref[...]


def _tc(body, out_shapes, *args):
    return pl.pallas_call(
        body,
        out_shape=[jax.ShapeDtypeStruct(s, _f32) for s in out_shapes],
        compiler_params=pltpu.CompilerParams(vmem_limit_bytes=100 * 2**20),
    )(*args)


def kernel(x, edge_index, batch, W0, b0, W1, b1, pool_w, Wd0, bd0, Wd1, bd1):
    src = edge_index[0].astype(jnp.int32)
    dst = edge_index[1].astype(jnp.int32)
    padi = jnp.arange(NW * PAD_PER_TILE, dtype=jnp.int32)
    padr = N + (padi % (NP - N))

    def pack_idx(a):
        a = jnp.concatenate(
            [a.reshape(NW, E // NW), padr.reshape(NW, PAD_PER_TILE)],
            axis=1)
        p = a[:, 0::2] | (a[:, 1::2] << 16)
        return p.reshape(NW * RPT // 2, 128)

    srcE = pack_idx(src)
    dstE = pack_idx(dst)

    xP = jnp.zeros((NP, D), _f32).at[:N].set(x.astype(_f32))
    ones_sel = jnp.zeros((NP,), _f32).at[:N].set(1.0)

    count_ones = _make_ones_count()
    count = _make_segsum(1)
    segsum = _make_segsum(D)

    cnt = count_ones(dstE).reshape(NC, NP, 1)
    t0, dinv = _tc(_tc1, [(NP, D), (NP, 1)], cnt, xP, W0)
    S0 = segsum(t0, srcE, dstE)
    h0, sel80, xw1 = _tc(
        _tc2, [(NP, D), (NP // 128, 128), (NP, D)],
        S0, t0, dinv, b0.reshape(1, D), pool_w.reshape(1, D), W1)
    selN = sel80.reshape(NP)
    pcnt = count(selN, srcE, dstE).reshape(NC, NP, 1)
    t1, pdinv = _tc(_tc3, [(NP, D), (NP, 1)], pcnt, selN.reshape(NP, 1), xw1)
    S1 = segsum(t1, srcE, dstE)
    (t2,) = _tc(
        _tc4, [(NP, D)],
        S1, t1, pdinv, selN.reshape(NP, 1), b1.reshape(1, D),
        Wd0.reshape(2, D, D))
    S2 = segsum(t2, srcE, dstE)
    (t3,) = _tc(
        _tc5, [(NP, D)],
        S2, t2, pdinv, selN.reshape(NP, 1), bd0.reshape(1, D),
        Wd1.reshape(2, D, D), h0, dinv)
    S3 = segsum(t3, srcE, dstE)
    (out,) = _tc(_tc6, [(NP, D)], S3, t3, dinv, bd1.reshape(1, D))
    return out[:N]
